# baseline (speedup 1.0000x reference)
import functools

import jax
import jax.numpy as jnp
from jax import lax
from jax.experimental import pallas as pl
from jax.experimental.pallas import tpu as pltpu
from jax.experimental.pallas import tpu_sc as plsc

_NEG = float("-inf")
_L = 16

_SPLIT = 2560
_SC_LEN = 4096 - _SPLIT
_NSHARD = 8
_WROWS = _SC_LEN // _NSHARD
_CHUNK = 48
_NCHUNK = _WROWS // _CHUNK

_SORT8_NET = (
    (0, 1), (2, 3), (4, 5), (6, 7),
    (0, 2), (1, 3), (4, 6), (5, 7),
    (1, 2), (5, 6),
    (0, 4), (1, 5), (2, 6), (3, 7),
    (2, 4), (3, 5),
    (1, 2), (3, 4), (5, 6),
)

_CLEAN8_NET = (
    (0, 4), (1, 5), (2, 6), (3, 7),
    (0, 2), (1, 3), (4, 6), (5, 7),
    (0, 1), (2, 3), (4, 5), (6, 7),
)


def _cmpex(v, i, j):
    hi = jnp.maximum(v[i], v[j])
    lo = jnp.minimum(v[i], v[j])
    v[i] = hi
    v[j] = lo


def _sort8_desc(v):
    v = list(v)
    for i, j in _SORT8_NET:
        _cmpex(v, i, j)
    return v


def _merge_top8(r, s):
    m = [jnp.maximum(r[i], s[7 - i]) for i in range(8)]
    for i, j in _CLEAN8_NET:
        _cmpex(m, i, j)
    return m


def _sc_topk(x):
    mesh = plsc.VectorSubcoreMesh(core_axis_name="c", subcore_axis_name="s")

    @functools.partial(
        pl.kernel,
        mesh=mesh,
        out_type=jax.ShapeDtypeStruct((4, _NSHARD, 8, 1024), jnp.float32),
        scratch_types=[
            pltpu.VMEM((2, _CHUNK, 1024), jnp.float32),
            pltpu.VMEM((8, 1024), jnp.float32),
            pltpu.SemaphoreType.DMA,
            pltpu.SemaphoreType.DMA,
        ],
    )
    def sc_topk(x_hbm, out_hbm, buf, rst, sem0, sem1):
        cid = lax.axis_index("c")
        sid = lax.axis_index("s")
        wid = sid * 2 + cid
        b = wid // _NSHARD
        shard = wid % _NSHARD
        row0 = _SPLIT + shard * _WROWS
        sems = (sem0, sem1)

        neg = jnp.full((_L,), _NEG, jnp.float32)

        def init_body(g, carry):
            for k in range(8):
                rst[k, pl.ds(g * _L, _L)] = neg
            return carry

        lax.fori_loop(0, 1024 // _L, init_body, 0)

        def start(chunk, slot):
            pltpu.make_async_copy(
                x_hbm.at[b, pl.ds(row0 + chunk * _CHUNK, _CHUNK), :],
                buf.at[slot],
                sems[slot],
            ).start()

        def wait(slot):
            pltpu.make_async_copy(
                x_hbm.at[b, pl.ds(0, _CHUNK), :],
                buf.at[slot],
                sems[slot],
            ).wait()

        def compute(slot):
            def grp_body(g, carry, _slot=slot):
                cb = g * _L
                r = tuple(rst[k, pl.ds(cb, _L)] for k in range(8))

                def blk_body(i, r, _cb=cb, _slot=_slot):
                    base = i * 8
                    v = [buf[_slot, base + j, pl.ds(_cb, _L)]
                         for j in range(8)]
                    v = _sort8_desc(v)
                    return tuple(_merge_top8(list(r), v))

                r = lax.fori_loop(0, _CHUNK // 8, blk_body, r)
                for k in range(8):
                    rst[k, pl.ds(cb, _L)] = r[k]
                return carry

            lax.fori_loop(0, 1024 // _L, grp_body, 0)

        start(0, 0)
        start(1, 1)

        def pair(i2, carry):
            wait(0)
            compute(0)
            start(2 * i2 + 2, 0)
            wait(1)
            compute(1)
            start(2 * i2 + 3, 1)
            return carry

        lax.fori_loop(0, _NCHUNK // 2 - 1, pair, 0)
        wait(0)
        compute(0)
        wait(1)
        compute(1)

        pltpu.sync_copy(rst, out_hbm.at[b, shard])

    return sc_topk(x)


def _tc_body(x_ref, o_ref):
    def body(g, r):
        r = list(r)
        base = g * 64
        v = [x_ref[0, pl.ds(base + 8 * i, 8), :] for i in range(8)]
        v = _sort8_desc(v)
        return tuple(_merge_top8(r, v))

    init = tuple(jnp.full((8, 128), _NEG, jnp.float32) for _ in range(8))
    r = lax.fori_loop(0, _SPLIT // 64, body, init)

    c = jnp.concatenate(list(r), axis=0)
    rows = lax.broadcasted_iota(jnp.int32, (64, 128), 0)
    outs = []
    for _ in range(8):
        m = jnp.max(c, axis=0, keepdims=True)
        outs.append(m)
        occ = c == m
        idx = jnp.where(occ, rows, 64)
        imin = jnp.min(idx, axis=0, keepdims=True)
        c = jnp.where(occ & (rows == imin), _NEG, c)
    o_ref[0] = jnp.concatenate(outs, axis=0)


def _tc_topk(x):
    return pl.pallas_call(
        _tc_body,
        grid=(4, 8),
        in_specs=[pl.BlockSpec((1, _SPLIT, 128), lambda b, cb: (b, 0, cb))],
        out_specs=pl.BlockSpec((1, 8, 128), lambda b, cb: (b, 0, cb)),
        out_shape=jax.ShapeDtypeStruct((4, 8, 1024), jnp.float32),
        compiler_params=pltpu.CompilerParams(
            dimension_semantics=("parallel", "parallel"),
        ),
    )(x)


def _merge_body(a_ref, b_ref, o_ref):
    nc = 8 + 8 * _NSHARD
    rows = lax.broadcasted_iota(jnp.int32, (nc, 1024), 0)
    for b in range(4):
        c = jnp.concatenate([a_ref[b], b_ref[b]], axis=0)
        outs = []
        for _ in range(8):
            m = jnp.max(c, axis=0, keepdims=True)
            outs.append(m)
            occ = c == m
            idx = jnp.where(occ, rows, nc)
            imin = jnp.min(idx, axis=0, keepdims=True)
            c = jnp.where(occ & (rows == imin), _NEG, c)
        s = jnp.concatenate(outs, axis=0)
        o_ref[b] = jnp.transpose(s, (1, 0))


def _merge_topk(a, b):
    return pl.pallas_call(
        _merge_body,
        out_shape=jax.ShapeDtypeStruct((4, 1024, 8), jnp.float32),
    )(a, b)


def kernel(x):
    out_sc = _sc_topk(x)
    out_tc = _tc_topk(x)
    out = _merge_topk(out_tc, out_sc.reshape(4, 8 * _NSHARD, 1024))
    return out.reshape(4, 8 * 1024)


# speedup vs baseline: 1.0482x; 1.0482x over previous
import functools

import jax
import jax.numpy as jnp
from jax import lax
from jax.experimental import pallas as pl
from jax.experimental.pallas import tpu as pltpu
from jax.experimental.pallas import tpu_sc as plsc

_NEG = float("-inf")
_L = 16

_SPLIT = 2560
_CHUNK = 384
_SC_LEN = 4096 - _SPLIT
_NCHUNK = _SC_LEN // _CHUNK

_SORT8_NET = (
    (0, 1), (2, 3), (4, 5), (6, 7),
    (0, 2), (1, 3), (4, 6), (5, 7),
    (1, 2), (5, 6),
    (0, 4), (1, 5), (2, 6), (3, 7),
    (2, 4), (3, 5),
    (1, 2), (3, 4), (5, 6),
)

_CLEAN8_NET = (
    (0, 4), (1, 5), (2, 6), (3, 7),
    (0, 2), (1, 3), (4, 6), (5, 7),
    (0, 1), (2, 3), (4, 5), (6, 7),
)


def _cmpex(v, i, j):
    hi = jnp.maximum(v[i], v[j])
    lo = jnp.minimum(v[i], v[j])
    v[i] = hi
    v[j] = lo


def _sort8_desc(v):
    v = list(v)
    for i, j in _SORT8_NET:
        _cmpex(v, i, j)
    return v


def _merge_top8(r, s):
    m = [jnp.maximum(r[i], s[7 - i]) for i in range(8)]
    for i, j in _CLEAN8_NET:
        _cmpex(m, i, j)
    return m


def _sc_topk(x):
    mesh = plsc.VectorSubcoreMesh(core_axis_name="c", subcore_axis_name="s")

    @functools.partial(
        pl.kernel,
        mesh=mesh,
        out_type=jax.ShapeDtypeStruct((4, 8, 1024), jnp.float32),
        scratch_types=[
            pltpu.VMEM((2, _CHUNK, 128), jnp.float32),
            pltpu.VMEM((8, 128), jnp.float32),
            pltpu.SemaphoreType.DMA,
            pltpu.SemaphoreType.DMA,
        ],
    )
    def sc_topk(x_hbm, out_hbm, buf, rst, sem0, sem1):
        cid = lax.axis_index("c")
        sid = lax.axis_index("s")
        wid = sid * 2 + cid
        b = wid // 8
        cbase = (wid % 8) * 128
        sems = (sem0, sem1)

        neg = jnp.full((_L,), _NEG, jnp.float32)
        for k in range(8):
            for g in range(8):
                rst[k, pl.ds(g * _L, _L)] = neg

        def start(chunk, slot):
            pltpu.make_async_copy(
                x_hbm.at[b, pl.ds(_SPLIT + chunk * _CHUNK, _CHUNK),
                         pl.ds(cbase, 128)],
                buf.at[slot],
                sems[slot],
            ).start()

        def wait(slot):
            pltpu.make_async_copy(
                x_hbm.at[b, pl.ds(0, _CHUNK), pl.ds(cbase, 128)],
                buf.at[slot],
                sems[slot],
            ).wait()

        def compute(slot):
            for g in range(8):
                r = tuple(rst[k, pl.ds(g * _L, _L)] for k in range(8))

                def blk_body(i, r, _g=g, _slot=slot):
                    for u in range(2):
                        base = (i * 2 + u) * 8
                        v = [buf[_slot, base + j, pl.ds(_g * _L, _L)]
                             for j in range(8)]
                        v = _sort8_desc(v)
                        r = tuple(_merge_top8(list(r), v))
                    return r

                r = lax.fori_loop(0, _CHUNK // 16, blk_body, r)
                for k in range(8):
                    rst[k, pl.ds(g * _L, _L)] = r[k]

        start(0, 0)
        start(1, 1)

        def pair(i2, carry):
            wait(0)
            compute(0)
            start(2 * i2 + 2, 0)
            wait(1)
            compute(1)
            start(2 * i2 + 3, 1)
            return carry

        lax.fori_loop(0, _NCHUNK // 2 - 1, pair, 0)
        wait(0)
        compute(0)
        wait(1)
        compute(1)

        pltpu.sync_copy(rst, out_hbm.at[b, :, pl.ds(cbase, 128)])

    return sc_topk(x)


def _tc_body(x_ref, o_ref):
    def body(g, r):
        for u in range(2):
            base = (g * 2 + u) * 64
            v = [x_ref[0, pl.ds(base + 8 * i, 8), :] for i in range(8)]
            v = _sort8_desc(v)
            r = tuple(_merge_top8(list(r), v))
        return r

    init = tuple(jnp.full((8, 128), _NEG, jnp.float32) for _ in range(8))
    r = lax.fori_loop(0, _SPLIT // 128, body, init)

    c = jnp.concatenate(list(r), axis=0)
    rows = lax.broadcasted_iota(jnp.int32, (64, 128), 0)
    outs = []
    for _ in range(8):
        m = jnp.max(c, axis=0, keepdims=True)
        outs.append(m)
        occ = c == m
        idx = jnp.where(occ, rows, 64)
        imin = jnp.min(idx, axis=0, keepdims=True)
        c = jnp.where(occ & (rows == imin), _NEG, c)
    o_ref[0] = jnp.concatenate(outs, axis=0)


def _tc_topk(x):
    return pl.pallas_call(
        _tc_body,
        grid=(4, 8),
        in_specs=[pl.BlockSpec((1, _SPLIT, 128), lambda b, cb: (b, 0, cb))],
        out_specs=pl.BlockSpec((1, 8, 128), lambda b, cb: (b, 0, cb)),
        out_shape=jax.ShapeDtypeStruct((4, 8, 1024), jnp.float32),
        compiler_params=pltpu.CompilerParams(
            dimension_semantics=("parallel", "parallel"),
        ),
    )(x)


def _merge_body(a_ref, b_ref, o_ref):
    c = jnp.concatenate([a_ref[0], b_ref[0]], axis=0)
    rows = lax.broadcasted_iota(jnp.int32, (16, 1024), 0)
    outs = []
    for _ in range(8):
        m = jnp.max(c, axis=0, keepdims=True)
        outs.append(m)
        occ = c == m
        idx = jnp.where(occ, rows, 16)
        imin = jnp.min(idx, axis=0, keepdims=True)
        c = jnp.where(occ & (rows == imin), _NEG, c)
    o_ref[0] = jnp.concatenate(outs, axis=0)


def _merge_topk(a, b):
    return pl.pallas_call(
        _merge_body,
        grid=(4,),
        in_specs=[
            pl.BlockSpec((1, 8, 1024), lambda i: (i, 0, 0)),
            pl.BlockSpec((1, 8, 1024), lambda i: (i, 0, 0)),
        ],
        out_specs=pl.BlockSpec((1, 8, 1024), lambda i: (i, 0, 0)),
        out_shape=jax.ShapeDtypeStruct((4, 8, 1024), jnp.float32),
    )(a, b)


def kernel(x):
    out_sc = _sc_topk(x)
    out_tc = _tc_topk(x)
    out = _merge_topk(out_tc, out_sc)
    return jnp.transpose(out, (0, 2, 1)).reshape(4, 8 * 1024)
